# baseline (speedup 1.0000x reference)
import functools

import jax
import jax.numpy as jnp
from jax import lax
from jax.experimental import pallas as pl
from jax.experimental.pallas import tpu as pltpu
from jax.experimental.pallas import tpu_sc as plsc

N = 10000
E = 320000
D = 128
H = 128
G = 64
C = 10

NP = 10240
CHUNK = 128
EPC = 2560
EP = EPC * CHUNK
NW = 32
KW = EPC // NW
ROWS_PER_TILE = NP // 16
DEGW = 128

BN = 1024
GRID = NP // BN


_MESH = plsc.VectorSubcoreMesh(core_axis_name="c", subcore_axis_name="s")


@functools.partial(
    pl.kernel,
    out_type=jax.ShapeDtypeStruct((2, NP, D), jnp.float32),
    mesh=_MESH,
    scratch_types=[
        pltpu.VMEM((KW, CHUNK), jnp.int32),
        pltpu.VMEM((KW, CHUNK), jnp.int32),
        pltpu.VMEM((CHUNK, D), jnp.float32),
        pltpu.VMEM_SHARED((NP, D), jnp.float32),
        pltpu.SemaphoreType.DMA,
    ],
)
def _sc_agg(x_hbm, src_hbm, dst_hbm, zeros_hbm, out_hbm,
            src_v, dst_v, rows_v, acc_sh, sem):
    cid = lax.axis_index("c")
    sid = lax.axis_index("s")
    wid = sid * 2 + cid
    pltpu.sync_copy(zeros_hbm.at[pl.ds(sid * ROWS_PER_TILE, ROWS_PER_TILE)],
                    acc_sh.at[pl.ds(sid * ROWS_PER_TILE, ROWS_PER_TILE)])
    pltpu.sync_copy(src_hbm.at[pl.ds(wid * KW, KW)], src_v)
    pltpu.sync_copy(dst_hbm.at[pl.ds(wid * KW, KW)], dst_v)
    plsc.subcore_barrier()

    def body(j, carry):
        pltpu.async_copy(x_hbm.at[src_v.at[j]], rows_v, sem).wait()
        pltpu.sync_copy(rows_v, acc_sh.at[dst_v.at[j]], add=True)
        return carry

    lax.fori_loop(0, KW, body, 0)
    plsc.subcore_barrier()
    pltpu.sync_copy(acc_sh.at[pl.ds(sid * ROWS_PER_TILE, ROWS_PER_TILE)],
                    out_hbm.at[cid, pl.ds(sid * ROWS_PER_TILE, ROWS_PER_TILE)])


@functools.partial(
    pl.kernel,
    out_type=jax.ShapeDtypeStruct((2, NP, DEGW), jnp.float32),
    mesh=_MESH,
    scratch_types=[
        pltpu.VMEM((KW, CHUNK), jnp.int32),
        pltpu.VMEM((CHUNK, DEGW), jnp.float32),
        pltpu.VMEM_SHARED((NP, DEGW), jnp.float32),
    ],
)
def _sc_deg(dst_hbm, ones_hbm, zeros_hbm, out_hbm, dst_v, ones_v, acc_sh):
    cid = lax.axis_index("c")
    sid = lax.axis_index("s")
    wid = sid * 2 + cid
    pltpu.sync_copy(zeros_hbm.at[pl.ds(sid * ROWS_PER_TILE, ROWS_PER_TILE)],
                    acc_sh.at[pl.ds(sid * ROWS_PER_TILE, ROWS_PER_TILE)])
    pltpu.sync_copy(dst_hbm.at[pl.ds(wid * KW, KW)], dst_v)
    pltpu.sync_copy(ones_hbm, ones_v)
    plsc.subcore_barrier()

    def body(j, carry):
        pltpu.sync_copy(ones_v, acc_sh.at[dst_v.at[j]], add=True)
        return carry

    lax.fori_loop(0, KW, body, 0)
    plsc.subcore_barrier()
    pltpu.sync_copy(acc_sh.at[pl.ds(sid * ROWS_PER_TILE, ROWS_PER_TILE)],
                    out_hbm.at[cid, pl.ds(sid * ROWS_PER_TILE, ROWS_PER_TILE)])


def _dense_mean_body(agg_ref, deg_ref, x_ref, wl_ref, wr_ref, b_ref, o_ref):
    agg = agg_ref[0] + agg_ref[1]
    deg = deg_ref[0] + deg_ref[1]
    mean = agg / jnp.maximum(deg, 1.0)
    acc = jnp.dot(mean, wl_ref[...], preferred_element_type=jnp.float32)
    acc = acc + jnp.dot(x_ref[...], wr_ref[...], preferred_element_type=jnp.float32)
    o_ref[...] = jnp.maximum(acc + b_ref[...], 0.0)


def _sage_dense(agg_p, deg_p, x, Wl, Wr, b):
    return pl.pallas_call(
        _dense_mean_body,
        grid=(GRID,),
        in_specs=[
            pl.BlockSpec((2, BN, D), lambda i: (0, i, 0)),
            pl.BlockSpec((2, BN, 1), lambda i: (0, i, 0)),
            pl.BlockSpec((BN, D), lambda i: (i, 0)),
            pl.BlockSpec((D, H), lambda i: (0, 0)),
            pl.BlockSpec((D, H), lambda i: (0, 0)),
            pl.BlockSpec((1, H), lambda i: (0, 0)),
        ],
        out_specs=pl.BlockSpec((BN, H), lambda i: (i, 0)),
        out_shape=jax.ShapeDtypeStruct((NP, H), jnp.float32),
    )(agg_p, deg_p, x, Wl, Wr, b)


def _dense2_body(a_ref, b_ref, wa_ref, wb_ref, bias_ref, o_ref):
    acc = jnp.dot(a_ref[...], wa_ref[...], preferred_element_type=jnp.float32)
    acc = acc + jnp.dot(b_ref[...], wb_ref[...], preferred_element_type=jnp.float32)
    o_ref[...] = jnp.maximum(acc + bias_ref[...], 0.0)


def _jk_dense(h1, h2, Wa, Wb, b):
    return pl.pallas_call(
        _dense2_body,
        grid=(GRID,),
        in_specs=[
            pl.BlockSpec((BN, H), lambda i: (i, 0)),
            pl.BlockSpec((BN, H), lambda i: (i, 0)),
            pl.BlockSpec((H, H), lambda i: (0, 0)),
            pl.BlockSpec((H, H), lambda i: (0, 0)),
            pl.BlockSpec((1, H), lambda i: (0, 0)),
        ],
        out_specs=pl.BlockSpec((BN, H), lambda i: (i, 0)),
        out_shape=jax.ShapeDtypeStruct((NP, H), jnp.float32),
    )(h1, h2, Wa, Wb, b)


def _pool_body(b_ref, h_ref, o_ref):
    @pl.when(pl.program_id(0) == 0)
    def _():
        o_ref[...] = jnp.zeros_like(o_ref)

    seg = b_ref[...]
    iota = lax.broadcasted_iota(jnp.int32, (BN, G), 1)
    mask = (seg == iota).astype(jnp.float32)
    o_ref[...] += lax.dot_general(mask, h_ref[...], (((0,), (0,)), ((), ())),
                                  preferred_element_type=jnp.float32)


def _pool(batch_p, h):
    return pl.pallas_call(
        _pool_body,
        grid=(GRID,),
        in_specs=[
            pl.BlockSpec((BN, 1), lambda i: (i, 0)),
            pl.BlockSpec((BN, H), lambda i: (i, 0)),
        ],
        out_specs=pl.BlockSpec((G, H), lambda i: (0, 0)),
        out_shape=jax.ShapeDtypeStruct((G, H), jnp.float32),
    )(batch_p, h)


def _head_body(z_ref, g_ref, be_ref, w1_ref, b1_ref, w2_ref, b2_ref, o_ref):
    z = z_ref[...]
    mu = jnp.mean(z, axis=0, keepdims=True)
    var = jnp.mean((z - mu) ** 2, axis=0, keepdims=True)
    zn = (z - mu) / jnp.sqrt(var + 1e-5) * g_ref[...] + be_ref[...]
    a = jnp.maximum(jnp.dot(zn, w1_ref[...], preferred_element_type=jnp.float32)
                    + b1_ref[...], 0.0)
    o = jnp.dot(a, w2_ref[...], preferred_element_type=jnp.float32) + b2_ref[...]
    o = o - jnp.max(o, axis=1, keepdims=True)
    e = jnp.exp(o)
    o_ref[...] = e / jnp.sum(e, axis=1, keepdims=True)


def _head(z, gamma, beta, W1, b1, W2, b2):
    return pl.pallas_call(
        _head_body,
        out_shape=jax.ShapeDtypeStruct((G, C), jnp.float32),
    )(z, gamma, beta, W1, b1, W2, b2)


def kernel(x, edge_index, batch,
           b1_s1_Wl, b1_s1_bl, b1_s1_Wr, b1_s1_br,
           b1_s2_Wl, b1_s2_bl, b1_s2_Wr, b1_s2_br,
           b1_lin_W, b1_lin_b,
           b2_s1_Wl, b2_s1_bl, b2_s1_Wr, b2_s1_br,
           b2_s2_Wl, b2_s2_bl, b2_s2_Wr, b2_s2_br,
           b2_lin_W, b2_lin_b,
           bn_gamma, bn_beta,
           lin1_W, lin1_b, lin2_W, lin2_b):
    src = edge_index[0]
    dst = edge_index[1]
    pad = EP - E
    src_p = jnp.concatenate([src, jnp.zeros((pad,), jnp.int32)]).reshape(EPC, CHUNK)
    dst_p = jnp.concatenate([dst, jnp.full((pad,), N, jnp.int32)]).reshape(EPC, CHUNK)
    x_pad = jnp.zeros((NP, D), jnp.float32).at[:N].set(x)
    zeros_nd = jnp.zeros((NP, D), jnp.float32)
    ones_cw = jnp.ones((CHUNK, DEGW), jnp.float32)
    batch_p = jnp.concatenate([batch, jnp.full((NP - N,), G, jnp.int32)]).reshape(NP, 1)

    deg_p = _sc_deg(dst_p, ones_cw, zeros_nd)[:, :, 0:1]

    def block(h_in, Wl1, bl1, Wr1, br1, Wl2, bl2, Wr2, br2, linW, linb):
        agg1 = _sc_agg(h_in, src_p, dst_p, zeros_nd)
        h1 = _sage_dense(agg1, deg_p, h_in, Wl1, Wr1, (bl1 + br1).reshape(1, H))
        agg2 = _sc_agg(h1, src_p, dst_p, zeros_nd)
        h2 = _sage_dense(agg2, deg_p, h1, Wl2, Wr2, (bl2 + br2).reshape(1, H))
        return _jk_dense(h1, h2, linW[:H], linW[H:], linb.reshape(1, H))

    hb1 = block(x_pad, b1_s1_Wl, b1_s1_bl, b1_s1_Wr, b1_s1_br,
                b1_s2_Wl, b1_s2_bl, b1_s2_Wr, b1_s2_br, b1_lin_W, b1_lin_b)
    x1 = _pool(batch_p, hb1)
    hb2 = block(hb1, b2_s1_Wl, b2_s1_bl, b2_s1_Wr, b2_s1_br,
                b2_s2_Wl, b2_s2_bl, b2_s2_Wr, b2_s2_br, b2_lin_W, b2_lin_b)
    x2 = _pool(batch_p, hb2)

    z = jnp.concatenate([x1, x2], axis=1)
    return _head(z, bn_gamma.reshape(1, 2 * H), bn_beta.reshape(1, 2 * H),
                 lin1_W, lin1_b.reshape(1, H), lin2_W, lin2_b.reshape(1, C))


# speedup vs baseline: 1.0494x; 1.0494x over previous
import functools

import jax
import jax.numpy as jnp
from jax import lax
from jax.experimental import pallas as pl
from jax.experimental.pallas import tpu as pltpu
from jax.experimental.pallas import tpu_sc as plsc

N = 10000
E = 320000
D = 128
H = 128
G = 64
C = 10

NP = 10240
CHUNK = 128
EPC = 2560
EP = EPC * CHUNK
NW = 32
KW = EPC // NW
ROWS_PER_TILE = NP // 16
DEGW = 128

BN = 1024
GRID = NP // BN


_MESH = plsc.VectorSubcoreMesh(core_axis_name="c", subcore_axis_name="s")


@functools.partial(
    pl.kernel,
    out_type=jax.ShapeDtypeStruct((2, NP, D), jnp.float32),
    mesh=_MESH,
    scratch_types=[
        pltpu.VMEM((KW, CHUNK), jnp.int32),
        pltpu.VMEM((KW, CHUNK), jnp.int32),
        pltpu.VMEM((CHUNK, D), jnp.float32),
        pltpu.VMEM_SHARED((NP, D), jnp.float32),
        pltpu.SemaphoreType.DMA,
    ],
)
def _sc_agg(x_hbm, src_hbm, dst_hbm, zeros_hbm, out_hbm,
            src_v, dst_v, rows_v, acc_sh, sem):
    cid = lax.axis_index("c")
    sid = lax.axis_index("s")
    wid = sid * 2 + cid
    pltpu.sync_copy(zeros_hbm.at[pl.ds(sid * ROWS_PER_TILE, ROWS_PER_TILE)],
                    acc_sh.at[pl.ds(sid * ROWS_PER_TILE, ROWS_PER_TILE)])
    pltpu.sync_copy(src_hbm.at[pl.ds(wid * KW, KW)], src_v)
    pltpu.sync_copy(dst_hbm.at[pl.ds(wid * KW, KW)], dst_v)
    plsc.subcore_barrier()

    def body(j, carry):
        pltpu.async_copy(x_hbm.at[src_v.at[j]], rows_v, sem).wait()
        pltpu.sync_copy(rows_v, acc_sh.at[dst_v.at[j]], add=True)
        return carry

    lax.fori_loop(0, KW, body, 0)
    plsc.subcore_barrier()
    pltpu.sync_copy(acc_sh.at[pl.ds(sid * ROWS_PER_TILE, ROWS_PER_TILE)],
                    out_hbm.at[cid, pl.ds(sid * ROWS_PER_TILE, ROWS_PER_TILE)])


@functools.partial(
    pl.kernel,
    out_type=jax.ShapeDtypeStruct((2, NP, DEGW), jnp.float32),
    mesh=_MESH,
    scratch_types=[
        pltpu.VMEM((KW, CHUNK), jnp.int32),
        pltpu.VMEM((CHUNK, DEGW), jnp.float32),
        pltpu.VMEM_SHARED((NP, DEGW), jnp.float32),
    ],
)
def _sc_deg(dst_hbm, ones_hbm, zeros_hbm, out_hbm, dst_v, ones_v, acc_sh):
    cid = lax.axis_index("c")
    sid = lax.axis_index("s")
    wid = sid * 2 + cid
    pltpu.sync_copy(zeros_hbm.at[pl.ds(sid * ROWS_PER_TILE, ROWS_PER_TILE)],
                    acc_sh.at[pl.ds(sid * ROWS_PER_TILE, ROWS_PER_TILE)])
    pltpu.sync_copy(dst_hbm.at[pl.ds(wid * KW, KW)], dst_v)
    pltpu.sync_copy(ones_hbm, ones_v)
    plsc.subcore_barrier()

    def body(j, carry):
        pltpu.sync_copy(ones_v, acc_sh.at[dst_v.at[j]], add=True)
        return carry

    lax.fori_loop(0, KW, body, 0)
    plsc.subcore_barrier()
    pltpu.sync_copy(acc_sh.at[pl.ds(sid * ROWS_PER_TILE, ROWS_PER_TILE)],
                    out_hbm.at[cid, pl.ds(sid * ROWS_PER_TILE, ROWS_PER_TILE)])


def _dense_mean_body(agg_ref, deg_ref, x_ref, wl_ref, wr_ref, b_ref, o_ref):
    agg = agg_ref[0] + agg_ref[1]
    deg = deg_ref[0] + deg_ref[1]
    mean = agg / jnp.maximum(deg, 1.0)
    acc = jnp.dot(mean, wl_ref[...], preferred_element_type=jnp.float32)
    acc = acc + jnp.dot(x_ref[...], wr_ref[...], preferred_element_type=jnp.float32)
    o_ref[...] = jnp.maximum(acc + b_ref[...], 0.0)


def _sage_dense(agg_p, deg_p, x, Wl, Wr, b):
    return pl.pallas_call(
        _dense_mean_body,
        grid=(GRID,),
        in_specs=[
            pl.BlockSpec((2, BN, D), lambda i: (0, i, 0)),
            pl.BlockSpec((2, BN, 1), lambda i: (0, i, 0)),
            pl.BlockSpec((BN, D), lambda i: (i, 0)),
            pl.BlockSpec((D, H), lambda i: (0, 0)),
            pl.BlockSpec((D, H), lambda i: (0, 0)),
            pl.BlockSpec((1, H), lambda i: (0, 0)),
        ],
        out_specs=pl.BlockSpec((BN, H), lambda i: (i, 0)),
        out_shape=jax.ShapeDtypeStruct((NP, H), jnp.float32),
    )(agg_p, deg_p, x, Wl, Wr, b)


def _dense2_body(a_ref, b_ref, wa_ref, wb_ref, bias_ref, o_ref):
    acc = jnp.dot(a_ref[...], wa_ref[...], preferred_element_type=jnp.float32)
    acc = acc + jnp.dot(b_ref[...], wb_ref[...], preferred_element_type=jnp.float32)
    o_ref[...] = jnp.maximum(acc + bias_ref[...], 0.0)


def _jk_dense(h1, h2, Wa, Wb, b):
    return pl.pallas_call(
        _dense2_body,
        grid=(GRID,),
        in_specs=[
            pl.BlockSpec((BN, H), lambda i: (i, 0)),
            pl.BlockSpec((BN, H), lambda i: (i, 0)),
            pl.BlockSpec((H, H), lambda i: (0, 0)),
            pl.BlockSpec((H, H), lambda i: (0, 0)),
            pl.BlockSpec((1, H), lambda i: (0, 0)),
        ],
        out_specs=pl.BlockSpec((BN, H), lambda i: (i, 0)),
        out_shape=jax.ShapeDtypeStruct((NP, H), jnp.float32),
    )(h1, h2, Wa, Wb, b)


def _sage_jk_pool_body(agg_ref, deg_ref, h1_ref, wl_ref, wr_ref, b_ref,
                       w1_ref, w2_ref, bj_ref, batch_ref, hb_ref, p_ref):
    agg = agg_ref[0] + agg_ref[1]
    deg = deg_ref[0] + deg_ref[1]
    mean = agg / jnp.maximum(deg, 1.0)
    h1 = h1_ref[...]
    acc = jnp.dot(mean, wl_ref[...], preferred_element_type=jnp.float32)
    acc = acc + jnp.dot(h1, wr_ref[...], preferred_element_type=jnp.float32)
    h2 = jnp.maximum(acc + b_ref[...], 0.0)
    acc2 = jnp.dot(h1, w1_ref[...], preferred_element_type=jnp.float32)
    acc2 = acc2 + jnp.dot(h2, w2_ref[...], preferred_element_type=jnp.float32)
    hb = jnp.maximum(acc2 + bj_ref[...], 0.0)
    hb_ref[...] = hb

    @pl.when(pl.program_id(0) == 0)
    def _():
        p_ref[...] = jnp.zeros_like(p_ref)

    seg = batch_ref[...]
    iota = lax.broadcasted_iota(jnp.int32, (BN, G), 1)
    mask = (seg == iota).astype(jnp.float32)
    p_ref[...] += lax.dot_general(mask, hb, (((0,), (0,)), ((), ())),
                                  preferred_element_type=jnp.float32)


def _sage_jk_pool(agg_p, deg_p, h1, Wl, Wr, b, Wj1, Wj2, bj, batch_p):
    return pl.pallas_call(
        _sage_jk_pool_body,
        grid=(GRID,),
        in_specs=[
            pl.BlockSpec((2, BN, D), lambda i: (0, i, 0)),
            pl.BlockSpec((2, BN, 1), lambda i: (0, i, 0)),
            pl.BlockSpec((BN, D), lambda i: (i, 0)),
            pl.BlockSpec((D, H), lambda i: (0, 0)),
            pl.BlockSpec((D, H), lambda i: (0, 0)),
            pl.BlockSpec((1, H), lambda i: (0, 0)),
            pl.BlockSpec((H, H), lambda i: (0, 0)),
            pl.BlockSpec((H, H), lambda i: (0, 0)),
            pl.BlockSpec((1, H), lambda i: (0, 0)),
            pl.BlockSpec((BN, 1), lambda i: (i, 0)),
        ],
        out_specs=[
            pl.BlockSpec((BN, H), lambda i: (i, 0)),
            pl.BlockSpec((G, H), lambda i: (0, 0)),
        ],
        out_shape=[
            jax.ShapeDtypeStruct((NP, H), jnp.float32),
            jax.ShapeDtypeStruct((G, H), jnp.float32),
        ],
    )(agg_p, deg_p, h1, Wl, Wr, b, Wj1, Wj2, bj, batch_p)


def _pool_body(b_ref, h_ref, o_ref):
    @pl.when(pl.program_id(0) == 0)
    def _():
        o_ref[...] = jnp.zeros_like(o_ref)

    seg = b_ref[...]
    iota = lax.broadcasted_iota(jnp.int32, (BN, G), 1)
    mask = (seg == iota).astype(jnp.float32)
    o_ref[...] += lax.dot_general(mask, h_ref[...], (((0,), (0,)), ((), ())),
                                  preferred_element_type=jnp.float32)


def _pool(batch_p, h):
    return pl.pallas_call(
        _pool_body,
        grid=(GRID,),
        in_specs=[
            pl.BlockSpec((BN, 1), lambda i: (i, 0)),
            pl.BlockSpec((BN, H), lambda i: (i, 0)),
        ],
        out_specs=pl.BlockSpec((G, H), lambda i: (0, 0)),
        out_shape=jax.ShapeDtypeStruct((G, H), jnp.float32),
    )(batch_p, h)


def _head_body(z_ref, g_ref, be_ref, w1_ref, b1_ref, w2_ref, b2_ref, o_ref):
    z = z_ref[...]
    mu = jnp.mean(z, axis=0, keepdims=True)
    var = jnp.mean((z - mu) ** 2, axis=0, keepdims=True)
    zn = (z - mu) / jnp.sqrt(var + 1e-5) * g_ref[...] + be_ref[...]
    a = jnp.maximum(jnp.dot(zn, w1_ref[...], preferred_element_type=jnp.float32)
                    + b1_ref[...], 0.0)
    o = jnp.dot(a, w2_ref[...], preferred_element_type=jnp.float32) + b2_ref[...]
    o = o - jnp.max(o, axis=1, keepdims=True)
    e = jnp.exp(o)
    o_ref[...] = e / jnp.sum(e, axis=1, keepdims=True)


def _head(z, gamma, beta, W1, b1, W2, b2):
    return pl.pallas_call(
        _head_body,
        out_shape=jax.ShapeDtypeStruct((G, C), jnp.float32),
    )(z, gamma, beta, W1, b1, W2, b2)


def kernel(x, edge_index, batch,
           b1_s1_Wl, b1_s1_bl, b1_s1_Wr, b1_s1_br,
           b1_s2_Wl, b1_s2_bl, b1_s2_Wr, b1_s2_br,
           b1_lin_W, b1_lin_b,
           b2_s1_Wl, b2_s1_bl, b2_s1_Wr, b2_s1_br,
           b2_s2_Wl, b2_s2_bl, b2_s2_Wr, b2_s2_br,
           b2_lin_W, b2_lin_b,
           bn_gamma, bn_beta,
           lin1_W, lin1_b, lin2_W, lin2_b):
    src = edge_index[0]
    dst = edge_index[1]
    pad = EP - E
    src_p = jnp.concatenate([src, jnp.zeros((pad,), jnp.int32)]).reshape(EPC, CHUNK)
    dst_p = jnp.concatenate([dst, jnp.full((pad,), N, jnp.int32)]).reshape(EPC, CHUNK)
    x_pad = jnp.zeros((NP, D), jnp.float32).at[:N].set(x)
    zeros_nd = jnp.zeros((NP, D), jnp.float32)
    ones_cw = jnp.ones((CHUNK, DEGW), jnp.float32)
    batch_p = jnp.concatenate([batch, jnp.full((NP - N,), G, jnp.int32)]).reshape(NP, 1)

    deg_p = _sc_deg(dst_p, ones_cw, zeros_nd)[:, :, 0:1]

    def block(h_in, Wl1, bl1, Wr1, br1, Wl2, bl2, Wr2, br2, linW, linb):
        agg1 = _sc_agg(h_in, src_p, dst_p, zeros_nd)
        h1 = _sage_dense(agg1, deg_p, h_in, Wl1, Wr1, (bl1 + br1).reshape(1, H))
        agg2 = _sc_agg(h1, src_p, dst_p, zeros_nd)
        return _sage_jk_pool(agg2, deg_p, h1, Wl2, Wr2,
                             (bl2 + br2).reshape(1, H),
                             linW[:H], linW[H:], linb.reshape(1, H), batch_p)

    hb1, x1 = block(x_pad, b1_s1_Wl, b1_s1_bl, b1_s1_Wr, b1_s1_br,
                    b1_s2_Wl, b1_s2_bl, b1_s2_Wr, b1_s2_br, b1_lin_W, b1_lin_b)
    hb2, x2 = block(hb1, b2_s1_Wl, b2_s1_bl, b2_s1_Wr, b2_s1_br,
                    b2_s2_Wl, b2_s2_bl, b2_s2_Wr, b2_s2_br, b2_lin_W, b2_lin_b)

    z = jnp.concatenate([x1, x2], axis=1)
    return _head(z, bn_gamma.reshape(1, 2 * H), bn_beta.reshape(1, 2 * H),
                 lin1_W, lin1_b.reshape(1, H), lin2_W, lin2_b.reshape(1, C))
